# baseline (speedup 1.0000x reference)
import jax
import jax.numpy as jnp
from jax.experimental import pallas as pl

_HEADS = 4
_C_ATT = 64
_NEG_SLOPE = 0.2

_NODE_BLK = 1024
_EDGE_BLK = 4096


def _pad_rows(a, blk):
    n = a.shape[0]
    pn = ((n + blk - 1) // blk) * blk
    if pn == n:
        return a, n
    pad = [(0, pn - n)] + [(0, 0)] * (a.ndim - 1)
    return jnp.pad(a, pad), n


def _proj_kernel(x_ref, wg_ref, asd_ref, xh_ref, asd_out_ref):
    x = x_ref[...]
    xh_ref[...] = jnp.dot(x, wg_ref[...], preferred_element_type=jnp.float32)
    asd_out_ref[...] = jnp.dot(x, asd_ref[...], preferred_element_type=jnp.float32)


def _alpha_kernel(ea_ref, aep_ref, as_ref, ad_ref, out_ref):
    a = jnp.dot(ea_ref[...], aep_ref[...], preferred_element_type=jnp.float32)
    a = a + as_ref[...] + ad_ref[...]
    out_ref[...] = jnp.where(a > 0, a, _NEG_SLOPE * a)


def _gcn_kernel(h_ref, w_ref, out_ref):
    h = h_ref[...]
    x1 = jnp.where(h > 0, h, jnp.exp(jnp.minimum(h, 0.0)) - 1.0)
    out_ref[...] = jnp.dot(x1, w_ref[...], preferred_element_type=jnp.float32)


def _head_kernel(agg_ref, w_ref, b_ref, out_ref):
    a = jnp.maximum(agg_ref[...], 0.0)
    logits = jnp.dot(a, w_ref[...], preferred_element_type=jnp.float32) + b_ref[...]
    m = jnp.max(logits, axis=1, keepdims=True)
    s = logits - m
    lse = jnp.log(jnp.sum(jnp.exp(s), axis=1, keepdims=True))
    out_ref[...] = s - lse


def kernel(x, edge_index, edge_attr, W_gat, att_src, att_dst, W_e, att_e, b_gat, W_gcn, b_gcn, W_lin, b_lin):
    n = x.shape[0]
    f_in = x.shape[1]
    d_edge = edge_attr.shape[1]
    hc = _HEADS * _C_ATT
    src = edge_index[0]
    dst = edge_index[1]

    as_proj = (W_gat.reshape(f_in, _HEADS, _C_ATT) * att_src[None, :, :]).sum(-1)
    ad_proj = (W_gat.reshape(f_in, _HEADS, _C_ATT) * att_dst[None, :, :]).sum(-1)
    asd = jnp.concatenate([as_proj, ad_proj], axis=1)
    ae_proj = (W_e.reshape(d_edge, _HEADS, _C_ATT) * att_e[None, :, :]).sum(-1)

    xp, _ = _pad_rows(x, _NODE_BLK)
    pn = xp.shape[0]
    grid_n = pn // _NODE_BLK
    xh_p, asd_p = pl.pallas_call(
        _proj_kernel,
        grid=(grid_n,),
        in_specs=[
            pl.BlockSpec((_NODE_BLK, f_in), lambda i: (i, 0)),
            pl.BlockSpec((f_in, hc), lambda i: (0, 0)),
            pl.BlockSpec((f_in, 2 * _HEADS), lambda i: (0, 0)),
        ],
        out_specs=[
            pl.BlockSpec((_NODE_BLK, hc), lambda i: (i, 0)),
            pl.BlockSpec((_NODE_BLK, 2 * _HEADS), lambda i: (i, 0)),
        ],
        out_shape=[
            jax.ShapeDtypeStruct((pn, hc), jnp.float32),
            jax.ShapeDtypeStruct((pn, 2 * _HEADS), jnp.float32),
        ],
    )(xp, W_gat, asd)
    xh = xh_p[:n]
    a_s = asd_p[:n, :_HEADS]
    a_d = asd_p[:n, _HEADS:]

    deg_in = jax.ops.segment_sum(jnp.ones_like(dst, dtype=x.dtype), dst, num_segments=n)
    ea_mean = jax.ops.segment_sum(edge_attr, dst, num_segments=n) / jnp.clip(deg_in, 1.0)[:, None]
    loop = jnp.arange(n, dtype=dst.dtype)
    src_f = jnp.concatenate([src, loop])
    dst_f = jnp.concatenate([dst, loop])
    ea_f = jnp.concatenate([edge_attr, ea_mean], axis=0)

    as_g = a_s[src_f]
    ad_g = a_d[dst_f]
    ea_p, ef = _pad_rows(ea_f, _EDGE_BLK)
    as_p, _ = _pad_rows(as_g, _EDGE_BLK)
    ad_p, _ = _pad_rows(ad_g, _EDGE_BLK)
    pe = ea_p.shape[0]
    grid_e = pe // _EDGE_BLK
    alpha_p = pl.pallas_call(
        _alpha_kernel,
        grid=(grid_e,),
        in_specs=[
            pl.BlockSpec((_EDGE_BLK, d_edge), lambda i: (i, 0)),
            pl.BlockSpec((d_edge, _HEADS), lambda i: (0, 0)),
            pl.BlockSpec((_EDGE_BLK, _HEADS), lambda i: (i, 0)),
            pl.BlockSpec((_EDGE_BLK, _HEADS), lambda i: (i, 0)),
        ],
        out_specs=pl.BlockSpec((_EDGE_BLK, _HEADS), lambda i: (i, 0)),
        out_shape=jax.ShapeDtypeStruct((pe, _HEADS), jnp.float32),
    )(ea_p, ae_proj, as_p, ad_p)
    alpha = alpha_p[:ef]

    amax = jax.ops.segment_max(alpha, dst_f, num_segments=n)
    alpha = jnp.exp(alpha - amax[dst_f])
    denom = jax.ops.segment_sum(alpha, dst_f, num_segments=n)
    alpha = alpha / (denom[dst_f] + 1e-16)

    msg = xh.reshape(n, _HEADS, _C_ATT)[src_f] * alpha[:, :, None]
    h = jax.ops.segment_sum(msg, dst_f, num_segments=n).reshape(n, hc) + b_gat

    hp, _ = _pad_rows(h, _NODE_BLK)
    c_gcn = W_gcn.shape[1]
    hg_p = pl.pallas_call(
        _gcn_kernel,
        grid=(grid_n,),
        in_specs=[
            pl.BlockSpec((_NODE_BLK, hc), lambda i: (i, 0)),
            pl.BlockSpec((hc, c_gcn), lambda i: (0, 0)),
        ],
        out_specs=pl.BlockSpec((_NODE_BLK, c_gcn), lambda i: (i, 0)),
        out_shape=jax.ShapeDtypeStruct((pn, c_gcn), jnp.float32),
    )(hp, W_gcn)
    hg = hg_p[:n]

    deg = jax.ops.segment_sum(jnp.ones_like(dst_f, dtype=x.dtype), dst_f, num_segments=n)
    dinv = jnp.where(deg > 0, 1.0 / jnp.sqrt(deg), 0.0)
    norm = dinv[src_f] * dinv[dst_f]
    agg = jax.ops.segment_sum(hg[src_f] * norm[:, None], dst_f, num_segments=n) + b_gcn

    aggp, _ = _pad_rows(agg, _NODE_BLK)
    nclass = W_lin.shape[1]
    out_p = pl.pallas_call(
        _head_kernel,
        grid=(grid_n,),
        in_specs=[
            pl.BlockSpec((_NODE_BLK, c_gcn), lambda i: (i, 0)),
            pl.BlockSpec((c_gcn, nclass), lambda i: (0, 0)),
            pl.BlockSpec((1, nclass), lambda i: (0, 0)),
        ],
        out_specs=pl.BlockSpec((_NODE_BLK, nclass), lambda i: (i, 0)),
        out_shape=jax.ShapeDtypeStruct((pn, nclass), jnp.float32),
    )(aggp, W_lin, b_lin.reshape(1, nclass))
    return out_p[:n]
